# Initial kernel scaffold; baseline (speedup 1.0000x reference)
import jax
import jax.numpy as jnp
from jax.experimental import pallas as pl


def kernel(x):
    raise NotImplementedError("write your pallas kernel here")


# speedup vs baseline: 2.8153x; 2.8153x over previous
import functools

import jax
import jax.numpy as jnp
from jax import lax
from jax.experimental import pallas as pl
from jax.experimental.pallas import tpu as pltpu
from jax.experimental.pallas import tpu_sc as plsc

_NUM_CORES = 2
_NUM_SUBCORES = 16
_NUM_WORKERS = _NUM_CORES * _NUM_SUBCORES
_L = 16

_RADIX_BITS = 11
_NUM_BINS = 1 << _RADIX_BITS
_SHIFTS = (0, _RADIX_BITS, 2 * _RADIX_BITS)


def _vec(val):
  return lax.full((_L,), val, jnp.int32)


def _lsr(x, k):
  if k == 0:
    return x
  return lax.shift_right_logical(x, _vec(k))


def _to_sortable_bits(f):
  b = plsc.bitcast(f, jnp.int32)
  sgn = lax.shift_right_arithmetic(b, _vec(31))
  flip = lax.bitwise_or(sgn, _vec(-(2**31)))
  return lax.bitwise_xor(b, flip)


@functools.partial(jax.jit, static_argnames=())
def _argsort_rows(x):
  rows, n = x.shape
  assert rows % _NUM_WORKERS == 0 and n % _L == 0
  rows_per_worker = rows // _NUM_WORKERS
  num_chunks = n // _L
  hist_chunks = _NUM_BINS // _L

  mesh = plsc.VectorSubcoreMesh(
      core_axis_name="c", subcore_axis_name="s")

  def body(x_hbm, out_hbm, keyf, ping, pong, hist_a, hist_b):
    cid = lax.axis_index("c")
    sid = lax.axis_index("s")
    wid = sid * _NUM_CORES + cid

    def zero_hist(h):
      def zbody(j, _):
        h[pl.ds(j * _L, _L)] = _vec(0)
        return 0
      lax.fori_loop(0, hist_chunks, zbody, 0)

    def prefix_hist(h):
      def pbody(j, carry):
        v = h[pl.ds(j * _L, _L)]
        csum = plsc.cumsum(v)
        h[pl.ds(j * _L, _L)] = csum - v + carry
        return carry + jnp.sum(v)
      lax.fori_loop(0, hist_chunks, pbody, jnp.int32(-1))

    def pass0_hist():
      def hbody(j, _):
        sl = pl.ds(j * _L, _L)
        u = _to_sortable_bits(keyf[sl])
        keyf[sl] = plsc.bitcast(u, jnp.float32)
        d = lax.bitwise_and(u, _vec(_NUM_BINS - 1))
        occ, last = plsc.scan_count(d)
        plsc.addupdate_scatter(hist_a, [d], occ, mask=last)
        return 0
      lax.fori_loop(0, num_chunks, hbody, 0)

    def permute(src, dst, shift, hist_cur, hist_next, next_shift):
      iota = lax.iota(jnp.int32, _L)

      def cbody(j, _):
        sl = pl.ds(j * _L, _L)
        if src is None:
          v_idx = iota + j * _L
          u = plsc.bitcast(keyf[sl], jnp.int32)
        else:
          v_idx = src[sl]
          u = plsc.bitcast(plsc.load_gather(keyf, [v_idx]), jnp.int32)
        d = lax.bitwise_and(_lsr(u, shift), _vec(_NUM_BINS - 1))
        occ, last = plsc.scan_count(d)
        base = plsc.load_gather(hist_cur, [d])
        plsc.store_scatter(dst, [base + occ], v_idx)
        plsc.addupdate_scatter(hist_cur, [d], occ, mask=last)
        if hist_next is not None:
          d2 = lax.bitwise_and(_lsr(u, next_shift), _vec(_NUM_BINS - 1))
          occ2, last2 = plsc.scan_count(d2)
          plsc.addupdate_scatter(hist_next, [d2], occ2, mask=last2)
        return 0

      lax.fori_loop(0, num_chunks, cbody, 0)

    def row_body(r, _):
      row = wid * rows_per_worker + r
      pltpu.sync_copy(x_hbm.at[row], keyf)
      zero_hist(hist_a)
      pass0_hist()
      prefix_hist(hist_a)
      zero_hist(hist_b)
      permute(None, ping, _SHIFTS[0], hist_a, hist_b, _SHIFTS[1])
      prefix_hist(hist_b)
      zero_hist(hist_a)
      permute(ping, pong, _SHIFTS[1], hist_b, hist_a, _SHIFTS[2])
      prefix_hist(hist_a)
      permute(pong, ping, _SHIFTS[2], hist_a, None, None)
      pltpu.sync_copy(ping, out_hbm.at[row])
      return 0

    lax.fori_loop(0, rows_per_worker, row_body, 0)

  run = pl.kernel(
      body,
      out_type=jax.ShapeDtypeStruct((rows, n), jnp.int32),
      mesh=mesh,
      compiler_params=pltpu.CompilerParams(needs_layout_passes=False),
      scratch_types=[
          pltpu.VMEM((n,), jnp.float32),
          pltpu.VMEM((n,), jnp.int32),
          pltpu.VMEM((n,), jnp.int32),
          pltpu.VMEM((_NUM_BINS,), jnp.int32),
          pltpu.VMEM((_NUM_BINS,), jnp.int32),
      ],
  )
  return run(x)


def kernel(x):
  return _argsort_rows(x)
